# Initial kernel scaffold; baseline (speedup 1.0000x reference)
import functools

import jax
import jax.numpy as jnp
from jax import lax
from jax.experimental import pallas as pl
from jax.experimental.pallas import tpu as pltpu
from jax.experimental.pallas import tpu_sc as plsc

N_NODES = 50000
N_EDGES = 800000
D_IN = 65
HID = 64

NC, NS, LANES = 2, 16, 16
NW = NC * NS
IDX_W = 128
CHUNK = 512
N_IDX = CHUNK // IDX_W
E_PAD = 802816
EW = E_PAD // NW
N_CH = EW // CHUNK
N_CH2 = (E_PAD // NS) // CHUNK

NHALF = N_NODES // 2
ZR = 256
STRIPE = 1568
TRASH = 25088
ACC_ROWS = TRASH + 8

_mesh = plsc.VectorSubcoreMesh(core_axis_name="c", subcore_axis_name="s")


def _proj_body(x_ref, w_ref, o_ref):
    o_ref[...] = jnp.dot(x_ref[...], w_ref[...],
                         preferred_element_type=jnp.float32)


def _proj(x, W1):
    return pl.pallas_call(
        _proj_body,
        grid=(25,),
        in_specs=[pl.BlockSpec((2000, D_IN), lambda i: (i, 0)),
                  pl.BlockSpec((D_IN, HID), lambda i: (0, 0))],
        out_specs=pl.BlockSpec((2000, HID), lambda i: (i, 0)),
        out_shape=jax.ShapeDtypeStruct((N_NODES, HID), jnp.float32),
    )(x, W1)


@functools.partial(
    pl.kernel,
    mesh=_mesh,
    out_type=[jax.ShapeDtypeStruct((E_PAD, HID), jnp.float32),
              jax.ShapeDtypeStruct((E_PAD, HID), jnp.float32)],
    scratch_types=[pltpu.VMEM((N_IDX, IDX_W), jnp.int32),
                   pltpu.VMEM((N_IDX, IDX_W), jnp.int32),
                   pltpu.VMEM((CHUNK, HID), jnp.float32),
                   pltpu.VMEM((CHUNK, HID), jnp.float32),
                   pltpu.SemaphoreType.DMA,
                   pltpu.SemaphoreType.DMA],
)
def _gather_kernel(y_hbm, src_hbm, dst_hbm, gs_hbm, gd_hbm,
                   isv, idv, rs, rd, sem_i, sem_g):
    wid = lax.axis_index("s") * NC + lax.axis_index("c")

    @pl.loop(0, N_CH)
    def _(c):
        ch = wid * N_CH + c
        row0 = ch * N_IDX
        e0 = ch * CHUNK
        cp_s = pltpu.async_copy(src_hbm.at[pl.ds(row0, N_IDX)], isv, sem_i)
        cp_d = pltpu.async_copy(dst_hbm.at[pl.ds(row0, N_IDX)], idv, sem_i)
        cp_s.wait()
        cp_d.wait()
        cps = []
        for j in range(N_IDX):
            cps.append(pltpu.async_copy(
                y_hbm.at[isv.at[j]], rs.at[pl.ds(j * IDX_W, IDX_W)], sem_g))
            cps.append(pltpu.async_copy(
                y_hbm.at[idv.at[j]], rd.at[pl.ds(j * IDX_W, IDX_W)], sem_g))
        for cp in cps:
            cp.wait()
        pltpu.sync_copy(rs, gs_hbm.at[pl.ds(e0, CHUNK)])
        pltpu.sync_copy(rd, gd_hbm.at[pl.ds(e0, CHUNK)])


def _mlp_body(b1_ref, w2_ref, b2_ref, w3_ref, b3_ref, gs_ref, gd_ref, m_ref):
    i = pl.program_id(0)
    h = jnp.maximum(gs_ref[...] - gd_ref[...] + b1_ref[...], 0.0)
    h = jnp.dot(h, w2_ref[...], preferred_element_type=jnp.float32)
    h = jnp.maximum(h + b2_ref[...], 0.0)
    m = jnp.dot(h, w3_ref[...], preferred_element_type=jnp.float32)
    m = m + b3_ref[...]
    row = i * 4096 + lax.broadcasted_iota(jnp.int32, m.shape, 0)
    m_ref[...] = jnp.where(row < N_EDGES, m, 0.0)


def _mlp(b1, W2, b2, W3, b3, gs, gd):
    grid = E_PAD // 4096
    vspec = pl.BlockSpec((1, HID), lambda i: (0, 0))
    wspec = pl.BlockSpec((HID, HID), lambda i: (0, 0))
    espec = pl.BlockSpec((4096, HID), lambda i: (i, 0))
    return pl.pallas_call(
        _mlp_body,
        grid=(grid,),
        in_specs=[vspec, wspec, vspec, wspec, vspec, espec, espec],
        out_specs=espec,
        out_shape=jax.ShapeDtypeStruct((E_PAD, HID), jnp.float32),
    )(b1, W2, b2, W3, b3, gs, gd)


@functools.partial(
    pl.kernel,
    mesh=_mesh,
    out_type=jax.ShapeDtypeStruct((N_NODES, HID), jnp.float32),
    scratch_types=[pltpu.VMEM((N_IDX, IDX_W), jnp.int32),
                   pltpu.VMEM((CHUNK, HID), jnp.float32),
                   pltpu.VMEM((ZR, HID), jnp.float32),
                   pltpu.VMEM_SHARED((ACC_ROWS, HID), jnp.float32),
                   pltpu.SemaphoreType.DMA],
)
def _scatter_kernel(m_hbm, dst_hbm, out_hbm, idxv, rows, zv, acc, sem):
    cid = lax.axis_index("c")
    sid = lax.axis_index("s")

    @pl.loop(0, ZR)
    def _(r):
        for q in range(HID // LANES):
            zv[r, pl.ds(q * LANES, LANES)] = jnp.zeros((LANES,), jnp.float32)

    base_r = sid * STRIPE
    for k in range(STRIPE // ZR):
        pltpu.sync_copy(zv, acc.at[pl.ds(base_r + k * ZR, ZR)])
    rem = STRIPE % ZR
    if rem:
        pltpu.sync_copy(zv.at[pl.ds(0, rem)],
                        acc.at[pl.ds(base_r + (STRIPE // ZR) * ZR, rem)])
    plsc.subcore_barrier()

    node0 = cid * NHALF

    @pl.loop(0, N_CH2)
    def _(c):
        ch = sid * N_CH2 + c
        row0 = ch * N_IDX
        e0 = ch * CHUNK
        cp_i = pltpu.async_copy(dst_hbm.at[pl.ds(row0, N_IDX)], idxv, sem)
        cp_m = pltpu.async_copy(m_hbm.at[pl.ds(e0, CHUNK)], rows, sem)
        cp_i.wait()
        cp_m.wait()
        for j in range(N_IDX):
            for q in range(IDX_W // LANES):
                v = idxv[j, pl.ds(q * LANES, LANES)]
                loc = v - node0
                ok = (loc >= 0) & (loc < NHALF)
                idxv[j, pl.ds(q * LANES, LANES)] = jnp.where(
                    ok, loc, TRASH + (v & 7))
        for j in range(N_IDX):
            pltpu.sync_copy(rows.at[pl.ds(j * IDX_W, IDX_W)],
                            acc.at[idxv.at[j]], add=True)

    plsc.subcore_barrier()

    out0 = cid * NHALF

    @pl.when(sid < NS - 1)
    def _():
        pltpu.sync_copy(acc.at[pl.ds(sid * STRIPE, STRIPE)],
                        out_hbm.at[pl.ds(out0 + sid * STRIPE, STRIPE)])

    @pl.when(sid == NS - 1)
    def _():
        last = NHALF - (NS - 1) * STRIPE
        pltpu.sync_copy(acc.at[pl.ds((NS - 1) * STRIPE, last)],
                        out_hbm.at[pl.ds(out0 + (NS - 1) * STRIPE, last)])


def kernel(x, edge_index, W1, b1, W2, b2, W3, b3):
    src = edge_index[0]
    dst = edge_index[1]
    pad = E_PAD - N_EDGES
    src2d = jnp.concatenate(
        [src, jnp.zeros((pad,), jnp.int32)]).reshape(E_PAD // IDX_W, IDX_W)
    dst2d = jnp.concatenate(
        [dst, jnp.zeros((pad,), jnp.int32)]).reshape(E_PAD // IDX_W, IDX_W)

    y = _proj(x, W1)
    gs, gd = _gather_kernel(y, src2d, dst2d)
    m = _mlp(b1.reshape(1, HID), W2, b2.reshape(1, HID), W3,
             b3.reshape(1, HID), gs, gd)
    return _scatter_kernel(m, dst2d)


# speedup vs baseline: 2.5499x; 2.5499x over previous
import functools

import jax
import jax.numpy as jnp
from jax import lax
from jax.experimental import pallas as pl
from jax.experimental.pallas import tpu as pltpu
from jax.experimental.pallas import tpu_sc as plsc

N_NODES = 50000
N_EDGES = 800000
D_IN = 65
HID = 64

NC, NS, LANES = 2, 16, 16
NW = NC * NS
IDX_W = 128
CHUNK = 512
N_IDX = CHUNK // IDX_W
E_PAD = 802816
EW = E_PAD // NW
N_CH = EW // CHUNK
N_CH2 = (E_PAD // NS) // CHUNK

NHALF = N_NODES // 2
CHUNK_SC = 256
N_IDX_SC = CHUNK_SC // IDX_W
N_CH_SC = (E_PAD // NS) // CHUNK_SC
TRASH = NHALF
ACC_ROWS = TRASH + 8
STRIPE_Z = ACC_ROWS // NS
STRIPE_O = 1563
LAST_O = NHALF - (NS - 1) * STRIPE_O

_mesh = plsc.VectorSubcoreMesh(core_axis_name="c", subcore_axis_name="s")
_sc_params = pltpu.CompilerParams(use_tc_tiling_on_sc=False)


def _proj_body(x_ref, w_ref, o_ref):
    o_ref[...] = jnp.dot(x_ref[...], w_ref[...],
                         preferred_element_type=jnp.float32)


def _proj(x, W1):
    return pl.pallas_call(
        _proj_body,
        grid=(25,),
        in_specs=[pl.BlockSpec((2000, D_IN), lambda i: (i, 0)),
                  pl.BlockSpec((D_IN, HID), lambda i: (0, 0))],
        out_specs=pl.BlockSpec((2000, HID), lambda i: (i, 0)),
        out_shape=jax.ShapeDtypeStruct((N_NODES, HID), jnp.float32),
    )(x, W1)


@functools.partial(
    pl.kernel,
    mesh=_mesh,
    out_type=[jax.ShapeDtypeStruct((E_PAD, HID), jnp.float32),
              jax.ShapeDtypeStruct((E_PAD, HID), jnp.float32)],
    scratch_types=[pltpu.VMEM((N_IDX, IDX_W), jnp.int32),
                   pltpu.VMEM((N_IDX, IDX_W), jnp.int32),
                   pltpu.VMEM((CHUNK, HID), jnp.float32),
                   pltpu.VMEM((CHUNK, HID), jnp.float32),
                   pltpu.SemaphoreType.DMA,
                   pltpu.SemaphoreType.DMA],
    compiler_params=_sc_params,
)
def _gather_kernel(y_hbm, src_hbm, dst_hbm, gs_hbm, gd_hbm,
                   isv, idv, rs, rd, sem_i, sem_g):
    wid = lax.axis_index("s") * NC + lax.axis_index("c")

    @pl.loop(0, N_CH)
    def _(c):
        ch = wid * N_CH + c
        row0 = ch * N_IDX
        e0 = ch * CHUNK
        cp_s = pltpu.async_copy(src_hbm.at[pl.ds(row0, N_IDX)], isv, sem_i)
        cp_d = pltpu.async_copy(dst_hbm.at[pl.ds(row0, N_IDX)], idv, sem_i)
        cp_s.wait()
        cp_d.wait()
        cps = []
        for j in range(N_IDX):
            cps.append(pltpu.async_copy(
                y_hbm.at[isv.at[j]], rs.at[pl.ds(j * IDX_W, IDX_W)], sem_g))
            cps.append(pltpu.async_copy(
                y_hbm.at[idv.at[j]], rd.at[pl.ds(j * IDX_W, IDX_W)], sem_g))
        for cp in cps:
            cp.wait()
        pltpu.sync_copy(rs, gs_hbm.at[pl.ds(e0, CHUNK)])
        pltpu.sync_copy(rd, gd_hbm.at[pl.ds(e0, CHUNK)])


def _mlp_body(b1_ref, w2_ref, b2_ref, w3_ref, b3_ref, gs_ref, gd_ref, m_ref):
    i = pl.program_id(0)
    h = jnp.maximum(gs_ref[...] - gd_ref[...] + b1_ref[...], 0.0)
    h = jnp.dot(h, w2_ref[...], preferred_element_type=jnp.float32)
    h = jnp.maximum(h + b2_ref[...], 0.0)
    m = jnp.dot(h, w3_ref[...], preferred_element_type=jnp.float32)
    m = m + b3_ref[...]
    row = i * 4096 + lax.broadcasted_iota(jnp.int32, m.shape, 0)
    m_ref[...] = jnp.where(row < N_EDGES, m, 0.0)


def _mlp(b1, W2, b2, W3, b3, gs, gd):
    grid = E_PAD // 4096
    vspec = pl.BlockSpec((1, HID), lambda i: (0, 0))
    wspec = pl.BlockSpec((HID, HID), lambda i: (0, 0))
    espec = pl.BlockSpec((4096, HID), lambda i: (i, 0))
    return pl.pallas_call(
        _mlp_body,
        grid=(grid,),
        in_specs=[vspec, wspec, vspec, wspec, vspec, espec, espec],
        out_specs=espec,
        out_shape=jax.ShapeDtypeStruct((E_PAD, HID), jnp.float32),
    )(b1, W2, b2, W3, b3, gs, gd)


@functools.partial(
    pl.kernel,
    mesh=_mesh,
    out_type=jax.ShapeDtypeStruct((N_NODES, HID), jnp.float32),
    scratch_types=[pltpu.VMEM((N_IDX_SC, IDX_W), jnp.int32),
                   pltpu.VMEM((CHUNK_SC, HID), jnp.float32),
                   pltpu.VMEM_SHARED((ACC_ROWS, HID), jnp.float32),
                   pltpu.SemaphoreType.DMA],
    compiler_params=_sc_params,
)
def _scatter_kernel(m_hbm, dst_hbm, out_hbm, idxv, rows, acc, sem):
    cid = lax.axis_index("c")
    sid = lax.axis_index("s")

    @pl.loop(0, CHUNK_SC)
    def _(r):
        for q in range(HID // LANES):
            rows[r, pl.ds(q * LANES, LANES)] = jnp.zeros((LANES,), jnp.float32)

    base_r = sid * STRIPE_Z
    for k in range(STRIPE_Z // CHUNK_SC):
        pltpu.sync_copy(rows, acc.at[pl.ds(base_r + k * CHUNK_SC, CHUNK_SC)])
    rem = STRIPE_Z % CHUNK_SC
    if rem:
        pltpu.sync_copy(
            rows.at[pl.ds(0, rem)],
            acc.at[pl.ds(base_r + (STRIPE_Z // CHUNK_SC) * CHUNK_SC, rem)])
    plsc.subcore_barrier()

    node0 = cid * NHALF

    @pl.loop(0, N_CH_SC)
    def _(c):
        ch = sid * N_CH_SC + c
        row0 = ch * N_IDX_SC
        e0 = ch * CHUNK_SC
        cp_i = pltpu.async_copy(dst_hbm.at[pl.ds(row0, N_IDX_SC)], idxv, sem)
        cp_m = pltpu.async_copy(m_hbm.at[pl.ds(e0, CHUNK_SC)], rows, sem)
        cp_i.wait()
        cp_m.wait()
        for j in range(N_IDX_SC):
            for q in range(IDX_W // LANES):
                v = idxv[j, pl.ds(q * LANES, LANES)]
                loc = v - node0
                ok = (loc >= 0) & (loc < NHALF)
                idxv[j, pl.ds(q * LANES, LANES)] = jnp.where(
                    ok, loc, TRASH + (v & 7))
        for j in range(N_IDX_SC):
            pltpu.sync_copy(rows.at[pl.ds(j * IDX_W, IDX_W)],
                            acc.at[idxv.at[j]], add=True)

    plsc.subcore_barrier()

    out0 = cid * NHALF

    @pl.when(sid < NS - 1)
    def _():
        pltpu.sync_copy(acc.at[pl.ds(sid * STRIPE_O, STRIPE_O)],
                        out_hbm.at[pl.ds(out0 + sid * STRIPE_O, STRIPE_O)])

    @pl.when(sid == NS - 1)
    def _():
        pltpu.sync_copy(acc.at[pl.ds((NS - 1) * STRIPE_O, LAST_O)],
                        out_hbm.at[pl.ds(out0 + (NS - 1) * STRIPE_O, LAST_O)])


def kernel(x, edge_index, W1, b1, W2, b2, W3, b3):
    src = edge_index[0]
    dst = edge_index[1]
    pad = E_PAD - N_EDGES
    src2d = jnp.concatenate(
        [src, jnp.zeros((pad,), jnp.int32)]).reshape(E_PAD // IDX_W, IDX_W)
    dst2d = jnp.concatenate(
        [dst, jnp.zeros((pad,), jnp.int32)]).reshape(E_PAD // IDX_W, IDX_W)

    y = _proj(x, W1)
    gs, gd = _gather_kernel(y, src2d, dst2d)
    m = _mlp(b1.reshape(1, HID), W2, b2.reshape(1, HID), W3,
             b3.reshape(1, HID), gs, gd)
    return _scatter_kernel(m, dst2d)
